# baseline (speedup 1.0000x reference)
import functools

import jax
import jax.numpy as jnp
from jax.experimental import pallas as pl
from jax.experimental.pallas import tpu as pltpu

_TOPK = 128


def _stage(vr, vf_ref, st_ref, dv_ref, ds_ref, s_wr, s_rd, *, topk, nc):
    min32 = jnp.int32(-2147483648)
    r = vr.shape[0]

    s_all = s_rd[:]
    bits = jax.lax.bitcast_convert_type(s_all, jnp.int32)
    key = bits ^ ((bits >> 31) & jnp.int32(0x7FFFFFFF))

    def sstep(j, p):
        trial = p | (jnp.int32(1) << j)
        thresh = trial ^ min32
        cmp = key >= thresh[None, :, :]
        cnt = jnp.sum(cmp.astype(jnp.int32), axis=(0, 2))[:, None]
        return jnp.where(cnt >= topk, trial, p)

    def loop_body(t, p):
        vf_chunk = vf_ref[0, pl.ds(t * 128, 128), :]
        s_wr[t] = jax.lax.dot_general(
            vr, vf_chunk, (((1,), (1,)), ((), ())),
            preferred_element_type=jnp.float32)
        spi = 32 // nc
        for q in range(spi):
            p = sstep(31 - spi * t - q, p)
        return p

    p = jax.lax.fori_loop(0, nc, loop_body, jnp.zeros((r, 1), jnp.int32))
    thresh = p ^ min32
    mask = key >= thresh[None, :, :]
    edges = jnp.where(mask, s_all / (1.0 + jnp.abs(s_all)), 0.0)

    ds_ref[0, 0, 0, :] = jnp.sum(edges * st_ref[0], axis=(0, 2))
    edges2 = jnp.transpose(edges, (1, 0, 2)).reshape(r, nc * 128)
    dv_ref[0] = jax.lax.dot_general(
        edges2, vf_ref[0], (((1,), (0,)), ((), ())),
        preferred_element_type=jnp.float32)


def _body(vr_ref, vf_ref, st_ref, dv_ref, ds_ref, s0, s1, *, topk, nc):
    i = pl.program_id(1)
    vr = vr_ref[0]
    par = jax.lax.rem(i, 2)

    @pl.when(par == 0)
    def _():
        _stage(vr, vf_ref, st_ref, dv_ref, ds_ref, s0, s1,
               topk=topk, nc=nc)

    @pl.when(par == 1)
    def _():
        _stage(vr, vf_ref, st_ref, dv_ref, ds_ref, s1, s0,
               topk=topk, nc=nc)


@jax.jit
def kernel(val, state):
    b, n, d = val.shape
    r = min(256, n)
    nb = n // r
    nc = n // 128
    topk = min(_TOPK, n)

    grid = (b, nb + 1)
    dv, ds = pl.pallas_call(
        functools.partial(_body, topk=topk, nc=nc),
        grid=grid,
        in_specs=[
            pl.BlockSpec((1, r, d), lambda bi, i: (bi, jnp.minimum(i, nb - 1), 0)),
            pl.BlockSpec((1, n, d), lambda bi, i: (bi, 0, 0)),
            pl.BlockSpec((1, nc, 1, 128), lambda bi, i: (bi, 0, 0, 0)),
        ],
        out_specs=[
            pl.BlockSpec((1, r, d), lambda bi, i: (bi, jnp.maximum(i - 1, 0), 0)),
            pl.BlockSpec((1, 1, 1, r), lambda bi, i: (bi, jnp.maximum(i - 1, 0), 0, 0)),
        ],
        out_shape=[
            jax.ShapeDtypeStruct((b, n, d), jnp.float32),
            jax.ShapeDtypeStruct((b, nb, 1, r), jnp.float32),
        ],
        scratch_shapes=[
            pltpu.VMEM((nc, r, 128), jnp.float32),
            pltpu.VMEM((nc, r, 128), jnp.float32),
        ],
    )(val, val, state.reshape(b, nc, 1, 128))
    return ds.reshape(b, n), dv


# speedup vs baseline: 1.1221x; 1.1221x over previous
import functools

import jax
import jax.numpy as jnp
from jax.experimental import pallas as pl

_TOPK = 128


def _body(vr_ref, vf_ref, st_ref, dv_ref, ds_ref, *, topk):
    min32 = jnp.int32(-2147483648)
    vr = vr_ref[0]
    vf = vf_ref[0]
    s = jax.lax.dot_general(
        vr, vf, (((1,), (1,)), ((), ())),
        preferred_element_type=jnp.float32)

    bits = jax.lax.bitcast_convert_type(s, jnp.int32)
    key = bits ^ ((bits >> 31) & jnp.int32(0x7FFFFFFF))

    r = s.shape[0]

    def step(t, p):
        sh = 30 - 2 * t
        t1 = p | (jnp.int32(1) << sh)
        t2 = p | (jnp.int32(2) << sh)
        t3 = p | (jnp.int32(3) << sh)
        c1 = jnp.sum((key >= (t1 ^ min32)).astype(jnp.int32), axis=1,
                     keepdims=True)
        c2 = jnp.sum((key >= (t2 ^ min32)).astype(jnp.int32), axis=1,
                     keepdims=True)
        c3 = jnp.sum((key >= (t3 ^ min32)).astype(jnp.int32), axis=1,
                     keepdims=True)
        p = jnp.where(c1 >= topk, t1, p)
        p = jnp.where(c2 >= topk, t2, p)
        p = jnp.where(c3 >= topk, t3, p)
        return p

    p = jax.lax.fori_loop(0, 16, step, jnp.zeros((r, 1), jnp.int32))
    mask = key >= (p ^ min32)

    edges = jnp.where(mask, s / (1.0 + jnp.abs(s)), 0.0)
    ds_ref[0, 0, 0, :] = jnp.sum(edges * st_ref[0, 0, :][None, :], axis=1)
    dv_ref[0] = jax.lax.dot_general(
        edges, vf, (((1,), (0,)), ((), ())),
        preferred_element_type=jnp.float32)


@jax.jit
def kernel(val, state):
    b, n, d = val.shape
    r = min(256, n)
    nb = n // r
    topk = min(_TOPK, n)

    grid = (b, nb)
    dv, ds = pl.pallas_call(
        functools.partial(_body, topk=topk),
        grid=grid,
        in_specs=[
            pl.BlockSpec((1, r, d), lambda bi, i: (bi, i, 0)),
            pl.BlockSpec((1, n, d), lambda bi, i: (bi, 0, 0)),
            pl.BlockSpec((1, 1, n), lambda bi, i: (bi, 0, 0)),
        ],
        out_specs=[
            pl.BlockSpec((1, r, d), lambda bi, i: (bi, i, 0)),
            pl.BlockSpec((1, 1, 1, r), lambda bi, i: (bi, i, 0, 0)),
        ],
        out_shape=[
            jax.ShapeDtypeStruct((b, n, d), jnp.float32),
            jax.ShapeDtypeStruct((b, nb, 1, r), jnp.float32),
        ],
    )(val, val, state.reshape(b, 1, n))
    return ds.reshape(b, n), dv
